# Initial kernel scaffold; baseline (speedup 1.0000x reference)
import jax
import jax.numpy as jnp
from jax.experimental import pallas as pl


def kernel(x, pos, batch, W1, b1, W2, b2, num_samples):
    raise NotImplementedError("write your pallas kernel here")


# speedup vs baseline: 17.4316x; 17.4316x over previous
import functools

import jax
import jax.numpy as jnp
import numpy as np
from jax import lax
from jax.experimental import pallas as pl
from jax.experimental.pallas import tpu as pltpu
from jax.experimental.pallas import tpu_sc as plsc

N = 10000
DF = 128
M = 5000
K = 16
RAD2 = np.float32(0.2 * 0.2)
DT = 128
BIG = np.int32(2 ** 30)

SC_NC, SC_NS = 2, 16
SC_NW = SC_NC * SC_NS
E_PAD = 81920
SC_CHUNK = 128
SC_CHUNKS = E_PAD // (SC_NW * SC_CHUNK)
SC_PER_W = E_PAD // SC_NW

QB_R = 256
G_R = 20
CH_R = 1024
NCH_R = (N + CH_R - 1) // CH_R
QB_M = 1000
F32 = jnp.float32
I32 = jnp.int32


def _fps_body(pos_ref, post_ref, sel_ref, posq_ref, mind_ref):
    iota = (lax.broadcasted_iota(I32, (8, 1250), 0) * 1250
            + lax.broadcasted_iota(I32, (8, 1250), 1))
    px_pl = post_ref[0]
    py_pl = post_ref[1]
    pz_pl = post_ref[2]

    def dist_to(row):
        dx = px_pl - row[0:1, 0:1]
        dy = py_pl - row[0:1, 1:2]
        dz = pz_pl - row[0:1, 2:3]
        return (dx * dx + dz * dz) + dy * dy

    row0 = pos_ref[0:1, :]
    mind_ref[:] = dist_to(row0)
    sel_ref[0:1, 0:1] = jnp.zeros((1, 1), I32)

    def body(i, last):
        row = pos_ref[pl.ds(last, 1), :]
        posq_ref[pl.ds(i - 1, 1), :] = row
        mind = jnp.minimum(mind_ref[:], dist_to(row))
        mind_ref[:] = mind
        m = jnp.max(mind)
        nxt = jnp.min(jnp.where(mind == m, iota, N))
        sel_ref[pl.ds(i, 1), 0:1] = jnp.broadcast_to(nxt, (1, 1))
        return nxt

    last = lax.fori_loop(1, M, body, jnp.array(0, I32))
    posq_ref[pl.ds(M - 1, 1), :] = pos_ref[pl.ds(last, 1), :]


def _radius_body(posq_ref, post_ref, col_ref, valid_ref):
    c = pl.program_id(1)

    @pl.when(c == 0)
    def _():
        col_ref[:] = jnp.full((QB_R, K), BIG, I32)

    done = jnp.max(col_ref[:, K - 1:K]) < BIG

    @pl.when(jnp.logical_not(done))
    def _():
        base = c * CH_R
        dx = posq_ref[:, 0:1] - post_ref[0:1, :]
        dz = posq_ref[:, 2:3] - post_ref[2:3, :]
        d2 = dx * dx + dz * dz
        dy = posq_ref[:, 1:2] - post_ref[1:2, :]
        d2 = d2 + dy * dy
        key = jnp.where(d2 <= RAD2,
                        base + lax.broadcasted_iota(I32, (1, CH_R), 1), BIG)
        lane16 = lax.broadcasted_iota(I32, (1, K), 1)

        def slot(s, carry):
            col, cur = carry
            cand = jnp.min(jnp.where(key > cur, key, BIG), axis=1,
                           keepdims=True)
            col = jnp.where((lane16 == s) & (col == BIG), cand, col)
            cur = jnp.max(jnp.where(lane16 == s, col, -1), axis=1,
                          keepdims=True)
            return col, cur

        col0 = col_ref[:]
        cur0 = jnp.full((QB_R, 1), -1, I32)
        col, _ = lax.fori_loop(0, K, slot, (col0, cur0))
        col_ref[:] = col

    @pl.when(c == NCH_R - 1)
    def _():
        col = col_ref[:]
        found = col < N
        col_ref[:] = jnp.where(found, col, 0)
        valid_ref[:] = found.astype(F32)


def _table_body(x_ref, pos_ref, w1a_ref, w1p_ref, t_ref):
    u = jnp.dot(x_ref[:], w1a_ref[:], preferred_element_type=F32)
    p = pos_ref[:]
    for c in range(3):
        u = u + p[:, c:c + 1] * w1p_ref[c:c + 1, :]
    t_ref[:] = u


@functools.lru_cache(maxsize=1)
def _make_sc_gather_kernel():
    mesh = plsc.VectorSubcoreMesh(core_axis_name="c", subcore_axis_name="s")

    @functools.partial(
        pl.kernel,
        mesh=mesh,
        out_type=jax.ShapeDtypeStruct((E_PAD, DT), jnp.float32),
        scratch_types=[
            pltpu.VMEM((SC_CHUNKS, SC_CHUNK), jnp.int32),
            pltpu.VMEM((SC_CHUNK, DT), jnp.float32),
            pltpu.SemaphoreType.DMA,
        ],
    )
    def _sc_gather_kernel(t_hbm, idx_hbm, out_hbm, idx_v, rows_v, sem):
        wid = lax.axis_index("s") * SC_NC + lax.axis_index("c")
        base = wid * SC_PER_W
        pltpu.sync_copy(idx_hbm.at[wid], idx_v)
        for j in range(SC_CHUNKS):
            pltpu.async_copy(t_hbm.at[idx_v.at[j]], rows_v, sem).wait()
            pltpu.sync_copy(rows_v,
                            out_hbm.at[pl.ds(base + j * SC_CHUNK, SC_CHUNK)])

    return _sc_gather_kernel


def _sc_gather(table, idx_flat_pad):
    k = _make_sc_gather_kernel()
    return k(table, idx_flat_pad.reshape(SC_NW, SC_CHUNKS, SC_CHUNK))


def _mlp_body(g_ref, t_ref, posq_ref, valid_ref, w1p_ref, b1_ref, w2_ref,
              b2_ref, out_ref):
    s = pl.program_id(1)

    pq = posq_ref[:]
    q1 = b1_ref[:] - (pq[:, 0:1] * w1p_ref[0:1, :]
                      + pq[:, 1:2] * w1p_ref[1:2, :]
                      + pq[:, 2:3] * w1p_ref[2:3, :])

    def edge_mlp(u):
        a = jnp.maximum(u + q1, 0.0)
        return jnp.dot(a, w2_ref[:], preferred_element_type=F32) + b2_ref[:]

    h2 = edge_mlp(g_ref[:])
    lane = lax.broadcasted_iota(I32, (1, K), 1)
    v = jnp.max(jnp.where(lane == s, valid_ref[:], 0.0), axis=1,
                keepdims=True)
    h2m = jnp.where(v > 0.5, h2, -jnp.inf)

    @pl.when(s == 0)
    def _():
        h2s = edge_mlp(t_ref[:])
        out_ref[:] = jnp.maximum(h2s, h2m)

    @pl.when(s != 0)
    def _():
        out_ref[:] = jnp.maximum(out_ref[:], h2m)


def kernel(x, pos, batch, W1, b1, W2, b2, num_samples):
    pos_t = pos.T
    post_planes = pos_t.reshape(3, 8, 1250)

    sel2d, pos_q = pl.pallas_call(
        _fps_body,
        out_shape=(jax.ShapeDtypeStruct((M, 1), I32),
                   jax.ShapeDtypeStruct((M, 3), F32)),
        scratch_shapes=[pltpu.VMEM((8, 1250), F32)],
    )(pos, post_planes)
    sel = sel2d[:, 0]

    col, valid = pl.pallas_call(
        _radius_body,
        grid=(G_R, NCH_R),
        in_specs=[pl.BlockSpec((QB_R, 3), lambda i, c: (i, 0)),
                  pl.BlockSpec((3, CH_R), lambda i, c: (0, c))],
        out_specs=(pl.BlockSpec((QB_R, K), lambda i, c: (i, 0)),
                   pl.BlockSpec((QB_R, K), lambda i, c: (i, 0))),
        out_shape=(jax.ShapeDtypeStruct((M, K), I32),
                   jax.ShapeDtypeStruct((M, K), F32)),
        compiler_params=pltpu.CompilerParams(
            dimension_semantics=("arbitrary", "arbitrary")),
    )(pos_q, pos_t)

    table = pl.pallas_call(
        _table_body,
        grid=(10,),
        in_specs=[pl.BlockSpec((1000, DF), lambda i: (i, 0)),
                  pl.BlockSpec((1000, 3), lambda i: (i, 0)),
                  pl.BlockSpec((DF, DF), lambda i: (0, 0)),
                  pl.BlockSpec((3, DF), lambda i: (0, 0))],
        out_specs=pl.BlockSpec((1000, DT), lambda i: (i, 0)),
        out_shape=jax.ShapeDtypeStruct((N, DT), F32),
    )(x, pos, W1[0:DF, :], W1[DF:DF + 3, :])

    idx_flat = jnp.concatenate(
        [col.T.reshape(-1), jnp.zeros((E_PAD - M * K,), I32)])
    g = _sc_gather(table, idx_flat)

    slot_ok = (jnp.arange(K, dtype=I32)[None, :] <
               jnp.asarray(num_samples, I32))
    validf = valid * slot_ok.astype(F32)

    out = pl.pallas_call(
        _mlp_body,
        grid=(M // QB_M, K),
        in_specs=[pl.BlockSpec((QB_M, DT), lambda qb, s: (s * (M // QB_M) + qb, 0)),
                  pl.BlockSpec((QB_M, DT), lambda qb, s: (qb, 0)),
                  pl.BlockSpec((QB_M, 3), lambda qb, s: (qb, 0)),
                  pl.BlockSpec((QB_M, K), lambda qb, s: (qb, 0)),
                  pl.BlockSpec((3, DF), lambda qb, s: (0, 0)),
                  pl.BlockSpec((1, DF), lambda qb, s: (0, 0)),
                  pl.BlockSpec((DF, 256), lambda qb, s: (0, 0)),
                  pl.BlockSpec((1, 256), lambda qb, s: (0, 0))],
        out_specs=pl.BlockSpec((QB_M, 256), lambda qb, s: (qb, 0)),
        out_shape=jax.ShapeDtypeStruct((M, 256), F32),
        compiler_params=pltpu.CompilerParams(
            dimension_semantics=("arbitrary", "arbitrary")),
    )(g, table, pos_q, validf, W1[DF:DF + 3, :], b1.reshape(1, DF), W2,
      b2.reshape(1, 256))

    return out, pos_q, batch[sel]


# speedup vs baseline: 18.3835x; 1.0546x over previous
import functools

import jax
import jax.numpy as jnp
import numpy as np
from jax import lax
from jax.experimental import pallas as pl
from jax.experimental.pallas import tpu as pltpu
from jax.experimental.pallas import tpu_sc as plsc

N = 10000
DF = 128
M = 5000
K = 16
RAD2 = np.float32(0.2 * 0.2)
DT = 128
BIG = np.int32(2 ** 30)

SC_NC, SC_NS = 2, 16
SC_NW = SC_NC * SC_NS
E_PAD = 81920
SC_CHUNK = 128
SC_CHUNKS = E_PAD // (SC_NW * SC_CHUNK)
SC_PER_W = E_PAD // SC_NW

QB_R = 256
G_R = 20
CH_R = 1024
NCH_R = (N + CH_R - 1) // CH_R
QB_M = 1000
F32 = jnp.float32
I32 = jnp.int32


FPS_W = 1280


def _fps_body(pos_ref, post_ref, sel_ref, posq_ref, mind_ref):
    iota = (lax.broadcasted_iota(I32, (8, FPS_W), 0) * FPS_W
            + lax.broadcasted_iota(I32, (8, FPS_W), 1))
    pad = iota >= N
    px_pl = post_ref[0]
    py_pl = post_ref[1]
    pz_pl = post_ref[2]
    neg_inf = jnp.float32(-jnp.inf)

    def dist_to(row):
        dx = px_pl - row[0:1, 0:1]
        dy = py_pl - row[0:1, 1:2]
        dz = pz_pl - row[0:1, 2:3]
        return (dx * dx + dz * dz) + dy * dy

    def argmax_first(mind):
        parts = [(mind[:, t * 128:(t + 1) * 128],
                  iota[:, t * 128:(t + 1) * 128]) for t in range(FPS_W // 128)]
        while len(parts) > 1:
            nxt_parts = []
            for j in range(0, len(parts) - 1, 2):
                (av, ai), (bv, bi) = parts[j], parts[j + 1]
                tb = (bv > av) | ((bv == av) & (bi < ai))
                nxt_parts.append((jnp.where(tb, bv, av),
                                  jnp.where(tb, bi, ai)))
            if len(parts) % 2:
                nxt_parts.append(parts[-1])
            parts = nxt_parts
        va, ia = parts[0]
        m = jnp.max(jnp.max(va, axis=0, keepdims=True), axis=1, keepdims=True)
        return jnp.min(jnp.where(va == m, ia, BIG))

    row0 = pos_ref[0:1, :]
    mind_ref[:] = jnp.where(pad, neg_inf, dist_to(row0))
    sel_ref[0:1, 0:1] = jnp.zeros((1, 1), I32)

    def body(i, last):
        row = pos_ref[pl.ds(last, 1), :]
        posq_ref[pl.ds(i - 1, 1), :] = row
        mind = jnp.minimum(mind_ref[:], dist_to(row))
        mind_ref[:] = mind
        nxt = argmax_first(mind)
        sel_ref[pl.ds(i, 1), 0:1] = jnp.broadcast_to(nxt, (1, 1))
        return nxt

    last = lax.fori_loop(1, M, body, jnp.array(0, I32))
    posq_ref[pl.ds(M - 1, 1), :] = pos_ref[pl.ds(last, 1), :]


def _radius_body(posq_ref, post_ref, col_ref, valid_ref):
    c = pl.program_id(1)

    @pl.when(c == 0)
    def _():
        col_ref[:] = jnp.full((QB_R, K), BIG, I32)

    done = jnp.max(col_ref[:, K - 1:K]) < BIG

    @pl.when(jnp.logical_not(done))
    def _():
        base = c * CH_R
        dx = posq_ref[:, 0:1] - post_ref[0:1, :]
        dz = posq_ref[:, 2:3] - post_ref[2:3, :]
        d2 = dx * dx + dz * dz
        dy = posq_ref[:, 1:2] - post_ref[1:2, :]
        d2 = d2 + dy * dy
        key = jnp.where(d2 <= RAD2,
                        base + lax.broadcasted_iota(I32, (1, CH_R), 1), BIG)
        lane16 = lax.broadcasted_iota(I32, (1, K), 1)

        def slot(s, carry):
            col, cur = carry
            cand = jnp.min(jnp.where(key > cur, key, BIG), axis=1,
                           keepdims=True)
            col = jnp.where((lane16 == s) & (col == BIG), cand, col)
            cur = jnp.max(jnp.where(lane16 == s, col, -1), axis=1,
                          keepdims=True)
            return col, cur

        col0 = col_ref[:]
        cur0 = jnp.full((QB_R, 1), -1, I32)
        col, _ = lax.fori_loop(0, K, slot, (col0, cur0))
        col_ref[:] = col

    @pl.when(c == NCH_R - 1)
    def _():
        col = col_ref[:]
        found = col < N
        col_ref[:] = jnp.where(found, col, 0)
        valid_ref[:] = found.astype(F32)


def _table_body(x_ref, pos_ref, w1a_ref, w1p_ref, t_ref):
    u = jnp.dot(x_ref[:], w1a_ref[:], preferred_element_type=F32)
    p = pos_ref[:]
    for c in range(3):
        u = u + p[:, c:c + 1] * w1p_ref[c:c + 1, :]
    t_ref[:] = u


@functools.lru_cache(maxsize=1)
def _make_sc_gather_kernel():
    mesh = plsc.VectorSubcoreMesh(core_axis_name="c", subcore_axis_name="s")

    @functools.partial(
        pl.kernel,
        mesh=mesh,
        out_type=jax.ShapeDtypeStruct((E_PAD, DT), jnp.float32),
        scratch_types=[
            pltpu.VMEM((SC_CHUNKS, SC_CHUNK), jnp.int32),
            pltpu.VMEM((SC_CHUNK, DT), jnp.float32),
            pltpu.SemaphoreType.DMA,
        ],
    )
    def _sc_gather_kernel(t_hbm, idx_hbm, out_hbm, idx_v, rows_v, sem):
        wid = lax.axis_index("s") * SC_NC + lax.axis_index("c")
        base = wid * SC_PER_W
        pltpu.sync_copy(idx_hbm.at[wid], idx_v)
        for j in range(SC_CHUNKS):
            pltpu.async_copy(t_hbm.at[idx_v.at[j]], rows_v, sem).wait()
            pltpu.sync_copy(rows_v,
                            out_hbm.at[pl.ds(base + j * SC_CHUNK, SC_CHUNK)])

    return _sc_gather_kernel


def _sc_gather(table, idx_flat_pad):
    k = _make_sc_gather_kernel()
    return k(table, idx_flat_pad.reshape(SC_NW, SC_CHUNKS, SC_CHUNK))


def _mlp_body(g_ref, t_ref, posq_ref, valid_ref, w1p_ref, b1_ref, w2_ref,
              b2_ref, out_ref):
    s = pl.program_id(1)

    pq = posq_ref[:]
    q1 = b1_ref[:] - (pq[:, 0:1] * w1p_ref[0:1, :]
                      + pq[:, 1:2] * w1p_ref[1:2, :]
                      + pq[:, 2:3] * w1p_ref[2:3, :])

    def edge_mlp(u):
        a = jnp.maximum(u + q1, 0.0)
        return jnp.dot(a, w2_ref[:], preferred_element_type=F32) + b2_ref[:]

    h2 = edge_mlp(g_ref[:])
    lane = lax.broadcasted_iota(I32, (1, K), 1)
    v = jnp.max(jnp.where(lane == s, valid_ref[:], 0.0), axis=1,
                keepdims=True)
    h2m = jnp.where(v > 0.5, h2, -jnp.inf)

    @pl.when(s == 0)
    def _():
        h2s = edge_mlp(t_ref[:])
        out_ref[:] = jnp.maximum(h2s, h2m)

    @pl.when(s != 0)
    def _():
        out_ref[:] = jnp.maximum(out_ref[:], h2m)


def kernel(x, pos, batch, W1, b1, W2, b2, num_samples):
    pos_t = pos.T
    post_planes = jnp.pad(pos_t, ((0, 0), (0, 8 * FPS_W - N))).reshape(
        3, 8, FPS_W)

    sel2d, pos_q = pl.pallas_call(
        _fps_body,
        out_shape=(jax.ShapeDtypeStruct((M, 1), I32),
                   jax.ShapeDtypeStruct((M, 3), F32)),
        scratch_shapes=[pltpu.VMEM((8, FPS_W), F32)],
    )(pos, post_planes)
    sel = sel2d[:, 0]

    col, valid = pl.pallas_call(
        _radius_body,
        grid=(G_R, NCH_R),
        in_specs=[pl.BlockSpec((QB_R, 3), lambda i, c: (i, 0)),
                  pl.BlockSpec((3, CH_R), lambda i, c: (0, c))],
        out_specs=(pl.BlockSpec((QB_R, K), lambda i, c: (i, 0)),
                   pl.BlockSpec((QB_R, K), lambda i, c: (i, 0))),
        out_shape=(jax.ShapeDtypeStruct((M, K), I32),
                   jax.ShapeDtypeStruct((M, K), F32)),
        compiler_params=pltpu.CompilerParams(
            dimension_semantics=("arbitrary", "arbitrary")),
    )(pos_q, pos_t)

    table = pl.pallas_call(
        _table_body,
        grid=(10,),
        in_specs=[pl.BlockSpec((1000, DF), lambda i: (i, 0)),
                  pl.BlockSpec((1000, 3), lambda i: (i, 0)),
                  pl.BlockSpec((DF, DF), lambda i: (0, 0)),
                  pl.BlockSpec((3, DF), lambda i: (0, 0))],
        out_specs=pl.BlockSpec((1000, DT), lambda i: (i, 0)),
        out_shape=jax.ShapeDtypeStruct((N, DT), F32),
    )(x, pos, W1[0:DF, :], W1[DF:DF + 3, :])

    idx_flat = jnp.concatenate(
        [col.T.reshape(-1), jnp.zeros((E_PAD - M * K,), I32)])
    g = _sc_gather(table, idx_flat)

    slot_ok = (jnp.arange(K, dtype=I32)[None, :] <
               jnp.asarray(num_samples, I32))
    validf = valid * slot_ok.astype(F32)

    out = pl.pallas_call(
        _mlp_body,
        grid=(M // QB_M, K),
        in_specs=[pl.BlockSpec((QB_M, DT), lambda qb, s: (s * (M // QB_M) + qb, 0)),
                  pl.BlockSpec((QB_M, DT), lambda qb, s: (qb, 0)),
                  pl.BlockSpec((QB_M, 3), lambda qb, s: (qb, 0)),
                  pl.BlockSpec((QB_M, K), lambda qb, s: (qb, 0)),
                  pl.BlockSpec((3, DF), lambda qb, s: (0, 0)),
                  pl.BlockSpec((1, DF), lambda qb, s: (0, 0)),
                  pl.BlockSpec((DF, 256), lambda qb, s: (0, 0)),
                  pl.BlockSpec((1, 256), lambda qb, s: (0, 0))],
        out_specs=pl.BlockSpec((QB_M, 256), lambda qb, s: (qb, 0)),
        out_shape=jax.ShapeDtypeStruct((M, 256), F32),
        compiler_params=pltpu.CompilerParams(
            dimension_semantics=("arbitrary", "arbitrary")),
    )(g, table, pos_q, validf, W1[DF:DF + 3, :], b1.reshape(1, DF), W2,
      b2.reshape(1, 256))

    return out, pos_q, batch[sel]


# speedup vs baseline: 18.4284x; 1.0024x over previous
import functools

import jax
import jax.numpy as jnp
import numpy as np
from jax import lax
from jax.experimental import pallas as pl
from jax.experimental.pallas import tpu as pltpu
from jax.experimental.pallas import tpu_sc as plsc

N = 10000
DF = 128
M = 5000
K = 16
RAD2 = np.float32(0.2 * 0.2)
DT = 128
BIG = np.int32(2 ** 30)

SC_NC, SC_NS = 2, 16
SC_NW = SC_NC * SC_NS
E_PAD = 81920
SC_CHUNK = 128
SC_CHUNKS = E_PAD // (SC_NW * SC_CHUNK)
SC_PER_W = E_PAD // SC_NW

QB_R = 256
G_R = 20
CH_R = 1024
NCH_R = (N + CH_R - 1) // CH_R
QB_M = 1000
F32 = jnp.float32
I32 = jnp.int32


FPS_W = 1280


def _fps_body(pos_ref, post_ref, sel_ref, posq_ref):
    iota = (lax.broadcasted_iota(I32, (8, FPS_W), 0) * FPS_W
            + lax.broadcasted_iota(I32, (8, FPS_W), 1))
    pad = iota >= N
    px_pl = post_ref[0]
    py_pl = post_ref[1]
    pz_pl = post_ref[2]
    neg_inf = jnp.float32(-jnp.inf)

    def dist_to(row):
        dx = px_pl - row[0:1, 0:1]
        dy = py_pl - row[0:1, 1:2]
        dz = pz_pl - row[0:1, 2:3]
        return (dx * dx + dz * dz) + dy * dy

    def argmax_first(mind):
        parts = [(mind[:, t * 128:(t + 1) * 128],
                  iota[:, t * 128:(t + 1) * 128]) for t in range(FPS_W // 128)]
        while len(parts) > 1:
            nxt_parts = []
            for j in range(0, len(parts) - 1, 2):
                (av, ai), (bv, bi) = parts[j], parts[j + 1]
                tb = (bv > av) | ((bv == av) & (bi < ai))
                nxt_parts.append((jnp.where(tb, bv, av),
                                  jnp.where(tb, bi, ai)))
            if len(parts) % 2:
                nxt_parts.append(parts[-1])
            parts = nxt_parts
        va, ia = parts[0]
        m = jnp.max(jnp.max(va, axis=0, keepdims=True), axis=1, keepdims=True)
        return jnp.min(jnp.where(va == m, ia, BIG))

    row0 = pos_ref[0:1, :]
    mind0 = jnp.where(pad, neg_inf, dist_to(row0))
    sel_ref[0:1, 0:1] = jnp.zeros((1, 1), I32)

    def body(i, carry):
        last, mind = carry
        row = pos_ref[pl.ds(last, 1), :]
        posq_ref[pl.ds(i - 1, 1), :] = row
        mind = jnp.minimum(mind, dist_to(row))
        nxt = argmax_first(mind)
        sel_ref[pl.ds(i, 1), 0:1] = jnp.broadcast_to(nxt, (1, 1))
        return nxt, mind

    last, _ = lax.fori_loop(1, M, body, (jnp.array(0, I32), mind0))
    posq_ref[pl.ds(M - 1, 1), :] = pos_ref[pl.ds(last, 1), :]


def _radius_body(posq_ref, post_ref, col_ref, valid_ref):
    c = pl.program_id(1)

    @pl.when(c == 0)
    def _():
        col_ref[:] = jnp.full((QB_R, K), BIG, I32)

    done = jnp.max(col_ref[:, K - 1:K]) < BIG

    @pl.when(jnp.logical_not(done))
    def _():
        base = c * CH_R
        dx = posq_ref[:, 0:1] - post_ref[0:1, :]
        dz = posq_ref[:, 2:3] - post_ref[2:3, :]
        d2 = dx * dx + dz * dz
        dy = posq_ref[:, 1:2] - post_ref[1:2, :]
        d2 = d2 + dy * dy
        key = jnp.where(d2 <= RAD2,
                        base + lax.broadcasted_iota(I32, (1, CH_R), 1), BIG)
        lane16 = lax.broadcasted_iota(I32, (1, K), 1)

        def slot(s, carry):
            col, cur = carry
            cand = jnp.min(jnp.where(key > cur, key, BIG), axis=1,
                           keepdims=True)
            col = jnp.where((lane16 == s) & (col == BIG), cand, col)
            cur = jnp.max(jnp.where(lane16 == s, col, -1), axis=1,
                          keepdims=True)
            return col, cur

        col0 = col_ref[:]
        cur0 = jnp.full((QB_R, 1), -1, I32)
        col, _ = lax.fori_loop(0, K, slot, (col0, cur0))
        col_ref[:] = col

    @pl.when(c == NCH_R - 1)
    def _():
        col = col_ref[:]
        found = col < N
        col_ref[:] = jnp.where(found, col, 0)
        valid_ref[:] = found.astype(F32)


def _table_body(x_ref, pos_ref, w1a_ref, w1p_ref, t_ref):
    u = jnp.dot(x_ref[:], w1a_ref[:], preferred_element_type=F32)
    p = pos_ref[:]
    for c in range(3):
        u = u + p[:, c:c + 1] * w1p_ref[c:c + 1, :]
    t_ref[:] = u


@functools.lru_cache(maxsize=1)
def _make_sc_gather_kernel():
    mesh = plsc.VectorSubcoreMesh(core_axis_name="c", subcore_axis_name="s")

    @functools.partial(
        pl.kernel,
        mesh=mesh,
        out_type=jax.ShapeDtypeStruct((E_PAD, DT), jnp.float32),
        scratch_types=[
            pltpu.VMEM((SC_CHUNKS, SC_CHUNK), jnp.int32),
            pltpu.VMEM((SC_CHUNK, DT), jnp.float32),
            pltpu.SemaphoreType.DMA,
        ],
    )
    def _sc_gather_kernel(t_hbm, idx_hbm, out_hbm, idx_v, rows_v, sem):
        wid = lax.axis_index("s") * SC_NC + lax.axis_index("c")
        base = wid * SC_PER_W
        pltpu.sync_copy(idx_hbm.at[wid], idx_v)
        for j in range(SC_CHUNKS):
            pltpu.async_copy(t_hbm.at[idx_v.at[j]], rows_v, sem).wait()
            pltpu.sync_copy(rows_v,
                            out_hbm.at[pl.ds(base + j * SC_CHUNK, SC_CHUNK)])

    return _sc_gather_kernel


def _sc_gather(table, idx_flat_pad):
    k = _make_sc_gather_kernel()
    return k(table, idx_flat_pad.reshape(SC_NW, SC_CHUNKS, SC_CHUNK))


def _mlp_body(g_ref, t_ref, posq_ref, valid_ref, w1p_ref, b1_ref, w2_ref,
              b2_ref, out_ref):
    s = pl.program_id(1)

    pq = posq_ref[:]
    q1 = b1_ref[:] - (pq[:, 0:1] * w1p_ref[0:1, :]
                      + pq[:, 1:2] * w1p_ref[1:2, :]
                      + pq[:, 2:3] * w1p_ref[2:3, :])

    def edge_mlp(u):
        a = jnp.maximum(u + q1, 0.0)
        return jnp.dot(a, w2_ref[:], preferred_element_type=F32) + b2_ref[:]

    h2 = edge_mlp(g_ref[:])
    lane = lax.broadcasted_iota(I32, (1, K), 1)
    v = jnp.max(jnp.where(lane == s, valid_ref[:], 0.0), axis=1,
                keepdims=True)
    h2m = jnp.where(v > 0.5, h2, -jnp.inf)

    @pl.when(s == 0)
    def _():
        h2s = edge_mlp(t_ref[:])
        out_ref[:] = jnp.maximum(h2s, h2m)

    @pl.when(s != 0)
    def _():
        out_ref[:] = jnp.maximum(out_ref[:], h2m)


def kernel(x, pos, batch, W1, b1, W2, b2, num_samples):
    pos_t = pos.T
    post_planes = jnp.pad(pos_t, ((0, 0), (0, 8 * FPS_W - N))).reshape(
        3, 8, FPS_W)

    sel2d, pos_q = pl.pallas_call(
        _fps_body,
        out_shape=(jax.ShapeDtypeStruct((M, 1), I32),
                   jax.ShapeDtypeStruct((M, 3), F32)),
    )(pos, post_planes)
    sel = sel2d[:, 0]

    col, valid = pl.pallas_call(
        _radius_body,
        grid=(G_R, NCH_R),
        in_specs=[pl.BlockSpec((QB_R, 3), lambda i, c: (i, 0)),
                  pl.BlockSpec((3, CH_R), lambda i, c: (0, c))],
        out_specs=(pl.BlockSpec((QB_R, K), lambda i, c: (i, 0)),
                   pl.BlockSpec((QB_R, K), lambda i, c: (i, 0))),
        out_shape=(jax.ShapeDtypeStruct((M, K), I32),
                   jax.ShapeDtypeStruct((M, K), F32)),
        compiler_params=pltpu.CompilerParams(
            dimension_semantics=("arbitrary", "arbitrary")),
    )(pos_q, pos_t)

    table = pl.pallas_call(
        _table_body,
        grid=(10,),
        in_specs=[pl.BlockSpec((1000, DF), lambda i: (i, 0)),
                  pl.BlockSpec((1000, 3), lambda i: (i, 0)),
                  pl.BlockSpec((DF, DF), lambda i: (0, 0)),
                  pl.BlockSpec((3, DF), lambda i: (0, 0))],
        out_specs=pl.BlockSpec((1000, DT), lambda i: (i, 0)),
        out_shape=jax.ShapeDtypeStruct((N, DT), F32),
    )(x, pos, W1[0:DF, :], W1[DF:DF + 3, :])

    idx_flat = jnp.concatenate(
        [col.T.reshape(-1), jnp.zeros((E_PAD - M * K,), I32)])
    g = _sc_gather(table, idx_flat)

    slot_ok = (jnp.arange(K, dtype=I32)[None, :] <
               jnp.asarray(num_samples, I32))
    validf = valid * slot_ok.astype(F32)

    out = pl.pallas_call(
        _mlp_body,
        grid=(M // QB_M, K),
        in_specs=[pl.BlockSpec((QB_M, DT), lambda qb, s: (s * (M // QB_M) + qb, 0)),
                  pl.BlockSpec((QB_M, DT), lambda qb, s: (qb, 0)),
                  pl.BlockSpec((QB_M, 3), lambda qb, s: (qb, 0)),
                  pl.BlockSpec((QB_M, K), lambda qb, s: (qb, 0)),
                  pl.BlockSpec((3, DF), lambda qb, s: (0, 0)),
                  pl.BlockSpec((1, DF), lambda qb, s: (0, 0)),
                  pl.BlockSpec((DF, 256), lambda qb, s: (0, 0)),
                  pl.BlockSpec((1, 256), lambda qb, s: (0, 0))],
        out_specs=pl.BlockSpec((QB_M, 256), lambda qb, s: (qb, 0)),
        out_shape=jax.ShapeDtypeStruct((M, 256), F32),
        compiler_params=pltpu.CompilerParams(
            dimension_semantics=("arbitrary", "arbitrary")),
    )(g, table, pos_q, validf, W1[DF:DF + 3, :], b1.reshape(1, DF), W2,
      b2.reshape(1, 256))

    return out, pos_q, batch[sel]
